# baseline (speedup 1.0000x reference)
import functools
import math

import jax
import jax.numpy as jnp
import numpy as np
from jax import lax
from jax.experimental import pallas as pl
from jax.experimental.pallas import tpu as pltpu
from jax.experimental.pallas import tpu_sc as plsc

H = 4
DH = 32
D = 128
L = 2
N = 25000
E = 300000

NB = 1000
NACC = 25088
ROWS_PER_SUB = NACC // 16
EB = 32
BLOCKS_PER_SUB = 588
E_PAD = 16 * BLOCKS_PER_SUB * EB
AW = 72


def _lin_relu_body(x_ref, w_ref, b_ref, o_ref):
    y = jnp.dot(x_ref[0], w_ref[0], preferred_element_type=jnp.float32)
    o_ref[...] = jnp.maximum(y + b_ref[0, 0], 0.0)[None]


def _lin_relu(x2, w2, b2):
    return pl.pallas_call(
        _lin_relu_body,
        grid=(2, N // NB),
        in_specs=[
            pl.BlockSpec((1, NB, D), lambda t, i: (t, i, 0)),
            pl.BlockSpec((1, D, D), lambda t, i: (t, 0, 0)),
            pl.BlockSpec((1, 1, D), lambda t, i: (t, 0, 0)),
        ],
        out_specs=pl.BlockSpec((1, NB, D), lambda t, i: (t, i, 0)),
        out_shape=jax.ShapeDtypeStruct((2, N, D), jnp.float32),
    )(x2, w2, b2.reshape(2, 1, D))


def _proj_body(x_ref, w_ref, b_ref, o_ref):
    o_ref[...] = (
        jnp.dot(x_ref[...], w_ref[0, 0], preferred_element_type=jnp.float32)
        + b_ref[0, 0, 0]
    )


def _proj(x, wcat, bcat, p):
    return pl.pallas_call(
        _proj_body,
        grid=(N // NB, 2, p),
        in_specs=[
            pl.BlockSpec((NB, D), lambda i, j, q: (i, 0)),
            pl.BlockSpec((1, 1, D, 64), lambda i, j, q: (q, j, 0, 0)),
            pl.BlockSpec((1, 1, 1, 64), lambda i, j, q: (q, j, 0, 0)),
        ],
        out_specs=pl.BlockSpec((NB, 64), lambda i, j, q: (q * 2 * (N // NB) + j * (N // NB) + i, 0)),
        out_shape=jax.ShapeDtypeStruct((p * 2 * N, 64), jnp.float32),
    )(x, wcat, bcat.reshape(p, 2, 1, 64))


def _norm_agg(a):
    num = jnp.concatenate([a[0, :, 0:64], a[1, :, 0:64]], axis=1)
    den = jnp.concatenate(
        [
            jnp.broadcast_to(a[0, :, 64:65], (NB, DH)),
            jnp.broadcast_to(a[0, :, 65:66], (NB, DH)),
            jnp.broadcast_to(a[1, :, 64:65], (NB, DH)),
            jnp.broadcast_to(a[1, :, 65:66], (NB, DH)),
        ],
        axis=1,
    )
    return num / (den + 1e-16)


def _post_body(n_rel, acc_refs, x_ref, wa_ref, ba_ref, beta_ref, o_ref):
    agg = _norm_agg(acc_refs[0][...])
    for a_ref in acc_refs[1:]:
        agg = agg + _norm_agg(a_ref[...])
    o = jnp.dot(jax.nn.gelu(agg), wa_ref[...], preferred_element_type=jnp.float32)
    o = o + ba_ref[0]
    beta = beta_ref[0, 0]
    o_ref[...] = beta * o + (1.0 - beta) * x_ref[...]


def _post(accs, x_old, wa, ba, beta):
    n_rel = len(accs)

    def body(*refs):
        _post_body(n_rel, refs[:n_rel], *refs[n_rel:])

    return pl.pallas_call(
        body,
        grid=(N // NB,),
        in_specs=[pl.BlockSpec((2, NB, AW), lambda i: (0, i, 0))] * n_rel
        + [
            pl.BlockSpec((NB, D), lambda i: (i, 0)),
            pl.BlockSpec((D, D), lambda i: (0, 0)),
            pl.BlockSpec((1, D), lambda i: (0, 0)),
            pl.BlockSpec((1, 1), lambda i: (0, 0)),
        ],
        out_specs=pl.BlockSpec((NB, D), lambda i: (i, 0)),
        out_shape=jax.ShapeDtypeStruct((N, D), jnp.float32),
    )(*accs, x_old, wa.reshape(D, D), ba.reshape(1, D), beta.reshape(1, 1))


def _make_sc_agg(rel_specs, q_sel, q_off):
    mesh = plsc.VectorSubcoreMesh(core_axis_name="c", subcore_axis_name="s")

    def body(src_hbm, dstg_hbm, dsts_hbm, tbl_p, tbl_a, out_hbm,
             raw0, ika0, iva0, iq0, is0,
             raw1, ika1, iva1, iq1, is1,
             kab0, vab0, qb0, kab1, vab1, qb1, wb, acc,
             sk0, sv0, sq0, sk1, sv1, sq1):
        c = lax.axis_index("c")
        s = lax.axis_index("s")
        i16 = lax.iota(jnp.int32, 16)
        bufs = (
            (raw0, ika0, iva0, iq0, is0, kab0, vab0, qb0, sk0, sv0, sq0),
            (raw1, ika1, iva1, iq1, is1, kab1, vab1, qb1, sk1, sv1, sq1),
        )

        def zrow(e, carry):
            for k in range((AW - 16) // 8 + 1):
                wb[e, pl.ds(k * 8, 16)] = jnp.zeros((16,), jnp.float32)
            return carry

        lax.fori_loop(0, EB, zrow, 0)

        def zcp(j, carry):
            pltpu.sync_copy(wb, acc.at[pl.ds(s * ROWS_PER_SUB + j * EB, EB)])
            return carry

        lax.fori_loop(0, ROWS_PER_SUB // EB, zcp, 0)
        plsc.subcore_barrier()

        gdn = lax.GatherDimensionNumbers(
            offset_dims=(), collapsed_slice_dims=(0,), start_index_map=(0,))

        def allsum(v):
            for k in (8, 4, 2, 1):
                idx = lax.iota(jnp.int32, 16) ^ k
                v = v + lax.gather(v, idx[:, None], gdn, (1,),
                                   mode=lax.GatherScatterMode.PROMISE_IN_BOUNDS)
            return v

        q_tbl = (tbl_p, tbl_a)[q_sel]
        for (erow, tbl_sel, ka_off, va_off) in rel_specs:
            tbl = (tbl_p, tbl_a)[tbl_sel]
            base = erow * E_PAD + s * (BLOCKS_PER_SUB * EB)

            def issue(g, bset, base=base, tbl=tbl, ka_off=ka_off, va_off=va_off):
                raw, ika, iva, iq, isx, kab, vab, qb, sk, sv, sq = bset
                off = base + g * EB
                pltpu.sync_copy(src_hbm.at[pl.ds(off, EB)], raw)
                for j in range(EB // 16):
                    v = raw[pl.ds(j * 16, 16)]
                    ika[pl.ds(j * 16, 16)] = v + (ka_off * 2 * N + c * N)
                    iva[pl.ds(j * 16, 16)] = v + (va_off * 2 * N + c * N)
                pltpu.sync_copy(dstg_hbm.at[pl.ds(off, EB)], raw)
                for j in range(EB // 16):
                    iq[pl.ds(j * 16, 16)] = raw[pl.ds(j * 16, 16)] + (q_off * 2 * N + c * N)
                pltpu.sync_copy(dsts_hbm.at[pl.ds(off, EB)], isx)
                pltpu.async_copy(tbl.at[ika], kab, sk)
                pltpu.async_copy(tbl.at[iva], vab, sv)
                pltpu.async_copy(q_tbl.at[iq], qb, sq)

            def consume(bset, tbl=tbl):
                raw, ika, iva, iq, isx, kab, vab, qb, sk, sv, sq = bset
                pltpu.make_async_copy(tbl.at[ika], kab, sk).wait()
                pltpu.make_async_copy(q_tbl.at[iq], qb, sq).wait()
                pltpu.make_async_copy(tbl.at[iva], vab, sv).wait()

                def edge_body(e):
                    q0 = qb[e, pl.ds(0, 16)]
                    q1 = qb[e, pl.ds(16, 16)]
                    q2 = qb[e, pl.ds(32, 16)]
                    q3 = qb[e, pl.ds(48, 16)]
                    k0 = kab[e, pl.ds(0, 16)]
                    k1 = kab[e, pl.ds(16, 16)]
                    k2 = kab[e, pl.ds(32, 16)]
                    k3 = kab[e, pl.ds(48, 16)]
                    ev0 = jnp.exp(allsum(q0 * k0 + q1 * k1))
                    ev1 = jnp.exp(allsum(q2 * k2 + q3 * k3))
                    wb[e, pl.ds(0, 16)] = ev0 * vab[e, pl.ds(0, 16)]
                    wb[e, pl.ds(16, 16)] = ev0 * vab[e, pl.ds(16, 16)]
                    wb[e, pl.ds(32, 16)] = ev1 * vab[e, pl.ds(32, 16)]
                    w3 = ev1 * vab[e, pl.ds(48, 16)]
                    wb[e, pl.ds(48, 16)] = w3
                    sh = lax.gather(w3, ((i16 + 8) & 15)[:, None], gdn, (1,),
                                    mode=lax.GatherScatterMode.PROMISE_IN_BOUNDS)
                    tail = jnp.where(i16 == 8, ev0, jnp.where(i16 == 9, ev1, sh))
                    wb[e, pl.ds(56, 16)] = tail

                plsc.parallel_loop(0, EB, 1, unroll=4)(edge_body)
                pltpu.sync_copy(wb, acc.at[isx], add=True)

            issue(0, bufs[0])
            issue(1, bufs[1])

            def pair(i, carry):
                for b in range(2):
                    blk_i = i * 2 + b
                    consume(bufs[b])

                    @pl.when(blk_i + 2 < BLOCKS_PER_SUB)
                    def _(blk_i=blk_i, b=b):
                        issue(blk_i + 2, bufs[b])
                return carry

            lax.fori_loop(0, BLOCKS_PER_SUB // 2, pair, 0)

        plsc.subcore_barrier()
        pltpu.sync_copy(
            acc.at[pl.ds(s * ROWS_PER_SUB, ROWS_PER_SUB)],
            out_hbm.at[c, pl.ds(s * ROWS_PER_SUB, ROWS_PER_SUB)],
        )

    idx_t = pltpu.VMEM((EB,), jnp.int32)
    row_t = pltpu.VMEM((EB, 64), jnp.float32)
    return pl.kernel(
        body,
        out_type=jax.ShapeDtypeStruct((2, NACC, AW), jnp.float32),
        mesh=mesh,
        compiler_params=pltpu.CompilerParams(use_tc_tiling_on_sc=False),
        scratch_types=[
            idx_t, idx_t, idx_t, idx_t, idx_t,
            idx_t, idx_t, idx_t, idx_t, idx_t,
            row_t, row_t, row_t, row_t, row_t, row_t,
            pltpu.VMEM((EB, AW), jnp.float32),
            pltpu.VMEM_SHARED((NACC, AW), jnp.float32),
            pltpu.SemaphoreType.DMA,
            pltpu.SemaphoreType.DMA,
            pltpu.SemaphoreType.DMA,
            pltpu.SemaphoreType.DMA,
            pltpu.SemaphoreType.DMA,
            pltpu.SemaphoreType.DMA,
        ],
    )


_sc_writes = _make_sc_agg(((0, 1, 1, 2),), q_sel=0, q_off=0)
_sc_cites = _make_sc_agg(((1, 0, 1, 2),), q_sel=0, q_off=0)
_sc_rev = _make_sc_agg(((2, 0, 3, 4),), q_sel=1, q_off=0)


def _blockdiag(mats):
    z = jnp.zeros((D, D), jnp.float32)
    for h in range(H):
        z = z.at[h * DH:(h + 1) * DH, h * DH:(h + 1) * DH].set(mats[h])
    return z


def _split_halves(w, b):
    return w.reshape(D, 2, 64).transpose(1, 0, 2), b.reshape(2, 64)


def kernel(x_paper, x_author, ei_writes, ei_cites, ei_rev, lin_in_W, lin_in_b,
           Wk, bk, Wq, bq, Wv, bv, Wa, ba, a_rel, m_rel, p_rel, skip):
    f32 = jnp.float32
    x_paper = x_paper.astype(f32)
    x_author = x_author.astype(f32)

    def pad_edges(ei):
        srcv = ei[0].astype(jnp.int32)
        dstv = ei[1].astype(jnp.int32)
        zpad = jnp.zeros((E_PAD - E,), jnp.int32)
        return (
            jnp.concatenate([srcv, zpad]),
            jnp.concatenate([dstv, zpad]),
            jnp.concatenate([dstv, jnp.full((E_PAD - E,), N, jnp.int32)]),
        )

    sw, gw, tw = pad_edges(ei_writes)
    sc_, gc, tc_ = pad_edges(ei_cites)
    sr, gr, tr = pad_edges(ei_rev)
    src_flat = jnp.concatenate([sw, sc_, sr])
    dstg_flat = jnp.concatenate([gw, gc, gr])
    dsts_flat = jnp.concatenate([tw, tc_, tr])

    xs = _lin_relu(
        jnp.stack([x_paper, x_author]),
        lin_in_W.astype(f32),
        lin_in_b.astype(f32),
    )
    xp, xa = xs[0], xs[1]

    scale = 1.0 / math.sqrt(DH)
    rel_src = (1, 0, 0)

    for l in range(L):
        wka, bka, wvm, bvm = [], [], [], []
        for r in range(3):
            st = rel_src[r]
            ablk = _blockdiag(a_rel[l, r] * (p_rel[l, r][:, None, None] * scale))
            mblk = _blockdiag(m_rel[l, r])
            wka.append(Wk[l, st] @ ablk)
            bka.append(bk[l, st] @ ablk)
            wvm.append(Wv[l, st] @ mblk)
            bvm.append(bv[l, st] @ mblk)

        wp = [(Wq[l, 0], bq[l, 0]), (wka[1], bka[1]), (wvm[1], bvm[1]),
              (wka[2], bka[2]), (wvm[2], bvm[2])]
        wa_ = [(Wq[l, 1], bq[l, 1]), (wka[0], bka[0]), (wvm[0], bvm[0])]

        wcat_p = jnp.stack([_split_halves(w, b)[0] for w, b in wp])
        bcat_p = jnp.stack([_split_halves(w, b)[1] for w, b in wp])
        wcat_a = jnp.stack([_split_halves(w, b)[0] for w, b in wa_])
        bcat_a = jnp.stack([_split_halves(w, b)[1] for w, b in wa_])

        tbl_p = _proj(xp, wcat_p, bcat_p, 5)
        tbl_a = _proj(xa, wcat_a, bcat_a, 3)

        acc_w = _sc_writes(src_flat, dstg_flat, dsts_flat, tbl_p, tbl_a)
        acc_c = _sc_cites(src_flat, dstg_flat, dsts_flat, tbl_p, tbl_a)
        acc_r = _sc_rev(src_flat, dstg_flat, dsts_flat, tbl_p, tbl_a)

        beta_p = jax.nn.sigmoid(skip[l, 0]).astype(f32)
        beta_a = jax.nn.sigmoid(skip[l, 1]).astype(f32)
        xp = _post([acc_w[:, :N], acc_c[:, :N]], xp, Wa[l, 0], ba[l, 0], beta_p)
        xa = _post([acc_r[:, :N]], xa, Wa[l, 1], ba[l, 1], beta_a)

    return xp, xa


# speedup vs baseline: 1.4519x; 1.4519x over previous
import math

import jax
import jax.numpy as jnp
from jax import lax
from jax.experimental import pallas as pl
from jax.experimental.pallas import tpu as pltpu
from jax.experimental.pallas import tpu_sc as plsc

H = 4
DH = 32
D = 128
L = 2
N = 25000
E = 300000

NB = 1000
NACC = 25088
ROWS_PER_SUB = NACC // 16
EB = 64
CS = 3
BLOCKS_PER_SUB = 294
E_PAD = 16 * BLOCKS_PER_SUB * EB
AW = 72


def _lin_relu_body(x_ref, w_ref, b_ref, o_ref):
    y = jnp.dot(x_ref[0], w_ref[0], preferred_element_type=jnp.float32)
    o_ref[...] = jnp.maximum(y + b_ref[0, 0], 0.0)[None]


def _lin_relu(x2, w2, b2):
    return pl.pallas_call(
        _lin_relu_body,
        grid=(2, N // NB),
        in_specs=[
            pl.BlockSpec((1, NB, D), lambda t, i: (t, i, 0)),
            pl.BlockSpec((1, D, D), lambda t, i: (t, 0, 0)),
            pl.BlockSpec((1, 1, D), lambda t, i: (t, 0, 0)),
        ],
        out_specs=pl.BlockSpec((1, NB, D), lambda t, i: (t, i, 0)),
        out_shape=jax.ShapeDtypeStruct((2, N, D), jnp.float32),
    )(x2, w2, b2.reshape(2, 1, D))


def _proj_body(x_ref, w_ref, b_ref, o_ref):
    o_ref[...] = (
        jnp.dot(x_ref[...], w_ref[0, 0], preferred_element_type=jnp.float32)
        + b_ref[0, 0, 0]
    )


def _proj(x, wcat, bcat, p, w):
    return pl.pallas_call(
        _proj_body,
        grid=(N // NB, 2, p),
        in_specs=[
            pl.BlockSpec((NB, D), lambda i, j, q: (i, 0)),
            pl.BlockSpec((1, 1, D, w), lambda i, j, q: (q, j, 0, 0)),
            pl.BlockSpec((1, 1, 1, w), lambda i, j, q: (q, j, 0, 0)),
        ],
        out_specs=pl.BlockSpec(
            (NB, w), lambda i, j, q: (q * 2 * (N // NB) + j * (N // NB) + i, 0)),
        out_shape=jax.ShapeDtypeStruct((p * 2 * N, w), jnp.float32),
    )(x, wcat, bcat.reshape(p, 2, 1, w))


def _norm_agg(a):
    num = jnp.concatenate([a[0, :, 0:64], a[1, :, 0:64]], axis=1)
    den = jnp.concatenate(
        [
            jnp.broadcast_to(a[0, :, 64:65], (NB, DH)),
            jnp.broadcast_to(a[0, :, 65:66], (NB, DH)),
            jnp.broadcast_to(a[1, :, 64:65], (NB, DH)),
            jnp.broadcast_to(a[1, :, 65:66], (NB, DH)),
        ],
        axis=1,
    )
    return num / (den + 1e-16)


def _post_body(n_rel, acc_refs, x_ref, wa_ref, ba_ref, beta_ref, o_ref):
    agg = _norm_agg(acc_refs[0][...])
    for a_ref in acc_refs[1:]:
        agg = agg + _norm_agg(a_ref[...])
    o = jnp.dot(jax.nn.gelu(agg), wa_ref[...], preferred_element_type=jnp.float32)
    o = o + ba_ref[0]
    beta = beta_ref[0, 0]
    o_ref[...] = beta * o + (1.0 - beta) * x_ref[...]


def _post(accs, x_old, wa, ba, beta):
    n_rel = len(accs)

    def body(*refs):
        _post_body(n_rel, refs[:n_rel], *refs[n_rel:])

    return pl.pallas_call(
        body,
        grid=(N // NB,),
        in_specs=[pl.BlockSpec((2, NB, AW), lambda i: (0, i, 0))] * n_rel
        + [
            pl.BlockSpec((NB, D), lambda i: (i, 0)),
            pl.BlockSpec((D, D), lambda i: (0, 0)),
            pl.BlockSpec((1, D), lambda i: (0, 0)),
            pl.BlockSpec((1, 1), lambda i: (0, 0)),
        ],
        out_specs=pl.BlockSpec((NB, D), lambda i: (i, 0)),
        out_shape=jax.ShapeDtypeStruct((N, D), jnp.float32),
    )(*accs, x_old, wa.reshape(D, D), ba.reshape(1, D), beta.reshape(1, 1))


def _make_sc_agg(erow, kv_off):
    mesh = plsc.VectorSubcoreMesh(core_axis_name="c", subcore_axis_name="s")

    def body(src_hbm, dstg_hbm, dsts_hbm, q_hbm, kv_hbm, out_hbm,
             ssrc, sdstg, sdsts, ikv, iq, isx, kvb, qb, wb, acc,
             sem_kv, sem_q):
        c = lax.axis_index("c")
        s = lax.axis_index("s")
        i16 = lax.iota(jnp.int32, 16)

        def zrow(e, carry):
            for k in range((AW - 16) // 8 + 1):
                wb[e, pl.ds(k * 8, 16)] = jnp.zeros((16,), jnp.float32)
            return carry

        lax.fori_loop(0, EB, zrow, 0)

        def zcp(j, carry):
            pltpu.sync_copy(wb, acc.at[pl.ds(s * ROWS_PER_SUB + j * EB, EB)])
            return carry

        lax.fori_loop(0, ROWS_PER_SUB // EB, zcp, 0)
        if ROWS_PER_SUB % EB:
            pltpu.sync_copy(
                wb.at[pl.ds(0, ROWS_PER_SUB % EB)],
                acc.at[pl.ds(s * ROWS_PER_SUB + (ROWS_PER_SUB // EB) * EB,
                             ROWS_PER_SUB % EB)],
            )
        plsc.subcore_barrier()

        gdn = lax.GatherDimensionNumbers(
            offset_dims=(), collapsed_slice_dims=(0,), start_index_map=(0,))

        def allsum(v):
            for k in (8, 4, 2, 1):
                idx = lax.iota(jnp.int32, 16) ^ k
                v = v + lax.gather(v, idx[:, None], gdn, (1,),
                                   mode=lax.GatherScatterMode.PROMISE_IN_BOUNDS)
            return v

        def edge_body(e):
            q0 = qb[e, pl.ds(0, 16)]
            q1 = qb[e, pl.ds(16, 16)]
            q2 = qb[e, pl.ds(32, 16)]
            q3 = qb[e, pl.ds(48, 16)]
            k0 = kvb[e, pl.ds(0, 16)]
            k1 = kvb[e, pl.ds(16, 16)]
            k2 = kvb[e, pl.ds(32, 16)]
            k3 = kvb[e, pl.ds(48, 16)]
            ev0 = jnp.exp(allsum(q0 * k0 + q1 * k1))
            ev1 = jnp.exp(allsum(q2 * k2 + q3 * k3))
            wb[e, pl.ds(0, 16)] = ev0 * kvb[e, pl.ds(64, 16)]
            wb[e, pl.ds(16, 16)] = ev0 * kvb[e, pl.ds(80, 16)]
            wb[e, pl.ds(32, 16)] = ev1 * kvb[e, pl.ds(96, 16)]
            w3 = ev1 * kvb[e, pl.ds(112, 16)]
            wb[e, pl.ds(48, 16)] = w3
            sh = lax.gather(w3, ((i16 + 8) & 15)[:, None], gdn, (1,),
                            mode=lax.GatherScatterMode.PROMISE_IN_BOUNDS)
            tail = jnp.where(i16 == 8, ev0, jnp.where(i16 == 9, ev1, sh))
            wb[e, pl.ds(56, 16)] = tail

        base = erow * E_PAD + s * (BLOCKS_PER_SUB * EB)
        kv_c = kv_off * 2 * N + c * N
        q_c = c * N

        def chunk(ch, carry):
            coff = base + ch * (CS * EB)
            pltpu.sync_copy(src_hbm.at[pl.ds(coff, CS * EB)], ssrc)
            pltpu.sync_copy(dstg_hbm.at[pl.ds(coff, CS * EB)], sdstg)
            pltpu.sync_copy(dsts_hbm.at[pl.ds(coff, CS * EB)], sdsts)
            for b in range(CS):
                for j in range(EB // 16):
                    sl = pl.ds(b * EB + j * 16, 16)
                    dl = pl.ds(j * 16, 16)
                    ikv[dl] = ssrc[sl] + kv_c
                    iq[dl] = sdstg[sl] + q_c
                    isx[dl] = sdsts[sl]
                cp_kv = pltpu.async_copy(kv_hbm.at[ikv], kvb, sem_kv)
                cp_q = pltpu.async_copy(q_hbm.at[iq], qb, sem_q)
                cp_kv.wait()
                cp_q.wait()
                plsc.parallel_loop(0, EB, 1, unroll=4)(edge_body)
                pltpu.sync_copy(wb, acc.at[isx], add=True)
            return carry

        lax.fori_loop(0, BLOCKS_PER_SUB // CS, chunk, 0)

        plsc.subcore_barrier()
        pltpu.sync_copy(
            acc.at[pl.ds(s * ROWS_PER_SUB, ROWS_PER_SUB)],
            out_hbm.at[c, pl.ds(s * ROWS_PER_SUB, ROWS_PER_SUB)],
        )

    return pl.kernel(
        body,
        out_type=jax.ShapeDtypeStruct((2, NACC, AW), jnp.float32),
        mesh=mesh,
        compiler_params=pltpu.CompilerParams(use_tc_tiling_on_sc=False),
        scratch_types=[
            pltpu.VMEM((CS * EB,), jnp.int32),
            pltpu.VMEM((CS * EB,), jnp.int32),
            pltpu.VMEM((CS * EB,), jnp.int32),
            pltpu.VMEM((EB,), jnp.int32),
            pltpu.VMEM((EB,), jnp.int32),
            pltpu.VMEM((EB,), jnp.int32),
            pltpu.VMEM((EB, 2 * 64), jnp.float32),
            pltpu.VMEM((EB, 64), jnp.float32),
            pltpu.VMEM((EB, AW), jnp.float32),
            pltpu.VMEM_SHARED((NACC, AW), jnp.float32),
            pltpu.SemaphoreType.DMA,
            pltpu.SemaphoreType.DMA,
        ],
    )


_sc_writes = _make_sc_agg(0, 0)
_sc_cites = _make_sc_agg(1, 0)
_sc_rev = _make_sc_agg(2, 1)


def _blockdiag(mats):
    z = jnp.zeros((D, D), jnp.float32)
    for h in range(H):
        z = z.at[h * DH:(h + 1) * DH, h * DH:(h + 1) * DH].set(mats[h])
    return z


def _halves(w, b):
    w2 = w.shape[1]
    return (w.reshape(D, 2, w2 // 2).transpose(1, 0, 2),
            b.reshape(2, w2 // 2))


def _kv_halves(wk, bk_, wv, bv_):
    wh = [jnp.concatenate([wk[:, c * 64:(c + 1) * 64],
                           wv[:, c * 64:(c + 1) * 64]], axis=1) for c in (0, 1)]
    bh = [jnp.concatenate([bk_[c * 64:(c + 1) * 64],
                           bv_[c * 64:(c + 1) * 64]]) for c in (0, 1)]
    return jnp.stack(wh), jnp.stack(bh)


def kernel(x_paper, x_author, ei_writes, ei_cites, ei_rev, lin_in_W, lin_in_b,
           Wk, bk, Wq, bq, Wv, bv, Wa, ba, a_rel, m_rel, p_rel, skip):
    f32 = jnp.float32
    x_paper = x_paper.astype(f32)
    x_author = x_author.astype(f32)

    def pad_edges(ei):
        srcv = ei[0].astype(jnp.int32)
        dstv = ei[1].astype(jnp.int32)
        zpad = jnp.zeros((E_PAD - E,), jnp.int32)
        return (
            jnp.concatenate([srcv, zpad]),
            jnp.concatenate([dstv, zpad]),
            jnp.concatenate([dstv, jnp.full((E_PAD - E,), N, jnp.int32)]),
        )

    sw, gw, tw = pad_edges(ei_writes)
    sc_, gc, tc_ = pad_edges(ei_cites)
    sr, gr, tr = pad_edges(ei_rev)
    src_flat = jnp.concatenate([sw, sc_, sr])
    dstg_flat = jnp.concatenate([gw, gc, gr])
    dsts_flat = jnp.concatenate([tw, tc_, tr])

    xs = _lin_relu(
        jnp.stack([x_paper, x_author]),
        lin_in_W.astype(f32),
        lin_in_b.astype(f32),
    )
    xp, xa = xs[0], xs[1]

    scale = 1.0 / math.sqrt(DH)
    rel_src = (1, 0, 0)

    for l in range(L):
        wka, bka, wvm, bvm = [], [], [], []
        for r in range(3):
            st = rel_src[r]
            ablk = _blockdiag(a_rel[l, r] * (p_rel[l, r][:, None, None] * scale))
            mblk = _blockdiag(m_rel[l, r])
            wka.append(Wk[l, st] @ ablk)
            bka.append(bk[l, st] @ ablk)
            wvm.append(Wv[l, st] @ mblk)
            bvm.append(bv[l, st] @ mblk)

        qw_p, qb_p = _halves(Wq[l, 0], bq[l, 0])
        qw_a, qb_a = _halves(Wq[l, 1], bq[l, 1])
        kvw_c, kvb_c = _kv_halves(wka[1], bka[1], wvm[1], bvm[1])
        kvw_r, kvb_r = _kv_halves(wka[2], bka[2], wvm[2], bvm[2])
        kvw_w, kvb_w = _kv_halves(wka[0], bka[0], wvm[0], bvm[0])

        q_p = _proj(xp, qw_p[None], qb_p[None], 1, 64)
        q_a = _proj(xa, qw_a[None], qb_a[None], 1, 64)
        kv_p = _proj(xp, jnp.stack([kvw_c, kvw_r]), jnp.stack([kvb_c, kvb_r]),
                     2, 128)
        kv_a = _proj(xa, kvw_w[None], kvb_w[None], 1, 128)

        acc_w = _sc_writes(src_flat, dstg_flat, dsts_flat, q_p, kv_a)
        acc_c = _sc_cites(src_flat, dstg_flat, dsts_flat, q_p, kv_p)
        acc_r = _sc_rev(src_flat, dstg_flat, dsts_flat, q_a, kv_p)

        beta_p = jax.nn.sigmoid(skip[l, 0]).astype(f32)
        beta_a = jax.nn.sigmoid(skip[l, 1]).astype(f32)
        xp = _post([acc_w[:, :N], acc_c[:, :N]], xp, Wa[l, 0], ba[l, 0], beta_p)
        xa = _post([acc_r[:, :N]], xa, Wa[l, 1], ba[l, 1], beta_a)

    return xp, xa
